# baseline (speedup 1.0000x reference)
import functools

import jax
import jax.numpy as jnp
from jax.experimental import pallas as pl

SPACING = 2
ROWS_PER_STEP = 1792


def _unpool_body(w_ref, h_ref, vh_ref, pooled_ref, sp_ref, out_ref):
    mrow = (pooled_ref[...] > 0.5).astype(jnp.float32)
    up = jnp.dot(mrow, w_ref[...],
                 preferred_element_type=jnp.float32)
    vhx = jnp.dot(vh_ref[...][:, :SPACING], h_ref[...],
                  preferred_element_type=jnp.float32)
    out_ref[...] = sp_ref[...] * up * vhx


@jax.jit
def _unpool(pooled_map, sparse_pattern, original_height, original_width):
    b, c, ph, pw = pooled_map.shape
    oh, ow = sparse_pattern.shape[2], sparse_pattern.shape[3]
    n = b * c
    n2 = n * ph
    lanes = SPACING * ow
    pooled2 = pooled_map.reshape(n2, pw)
    sp2 = sparse_pattern.reshape(n2, lanes)

    h_lim = jnp.asarray(original_height, jnp.int32)
    w_lim = jnp.asarray(original_width, jnp.int32)

    cc = jax.lax.broadcasted_iota(jnp.int32, (pw, lanes), 0)
    bb = jax.lax.broadcasted_iota(jnp.int32, (pw, lanes), 1)
    wmat = ((cc == (bb % ow) // SPACING) & ((bb % ow) < w_lim)).astype(
        jnp.float32)

    ss = jax.lax.broadcasted_iota(jnp.int32, (SPACING, lanes), 0)
    hmat = (ss == bb[:SPACING] // ow).astype(jnp.float32)

    ra = jax.lax.broadcasted_iota(jnp.int32, (n2, 8), 0) % ph
    sa = jax.lax.broadcasted_iota(jnp.int32, (n2, 8), 1)
    vh = ((SPACING * ra + sa < h_lim) & (sa < SPACING)).astype(jnp.float32)

    r = ROWS_PER_STEP
    assert n2 % r == 0 and r % ph == 0
    out = pl.pallas_call(
        _unpool_body,
        grid=(n2 // r,),
        in_specs=[
            pl.BlockSpec((pw, lanes), lambda i: (0, 0)),
            pl.BlockSpec((SPACING, lanes), lambda i: (0, 0)),
            pl.BlockSpec((r, 8), lambda i: (i, 0)),
            pl.BlockSpec((r, pw), lambda i: (i, 0)),
            pl.BlockSpec((r, lanes), lambda i: (i, 0)),
        ],
        out_specs=pl.BlockSpec((r, lanes), lambda i: (i, 0)),
        out_shape=jax.ShapeDtypeStruct((n2, lanes), pooled_map.dtype),
    )(wmat, hmat, vh, pooled2, sp2)
    return out.reshape(b, c, oh, ow)


def kernel(pooled_map, sparse_pattern, original_height, original_width):
    return _unpool(pooled_map, sparse_pattern, original_height, original_width)


# speedup vs baseline: 1.2761x; 1.2761x over previous
import functools

import jax
import jax.numpy as jnp
from jax import lax
from jax.experimental import pallas as pl
from jax.experimental.pallas import tpu as pltpu
from jax.experimental.pallas import tpu_sc as plsc

SPACING = 2
L = 16


def _clamp01(x):
    return jnp.minimum(jnp.maximum(x, 0.0), 1.0)


def _sc_body(n_per_w, nc, ph, pw, oh, ow,
             pooled_hbm, sp_hbm, hl_hbm, wl_hbm, out_hbm, dump_hbm,
             pooled_v, img_v, upm_v, hl_v, wl_v, sem_s, sem_o):
    wid = lax.axis_index("s") * nc + lax.axis_index("c")
    pltpu.sync_copy(hl_hbm, hl_v)
    pltpu.sync_copy(wl_hbm, wl_v)
    h_lim = jnp.max(hl_v[...])
    w_lim = jnp.max(wl_v[...])
    h_valid = jnp.minimum(jnp.maximum(h_lim, 0), oh)
    w_valid = jnp.minimum(jnp.maximum(w_lim, 0), ow)
    nvec = ow // L
    crows = oh // 2
    cp = ph // 2
    full_vecs = w_valid // L
    wfix_on = jnp.minimum(nvec - full_vecs, 1)
    wfixf = (full_vecs * L).astype(jnp.float32)
    lanef = lax.iota(jnp.int32, L).astype(jnp.float32)
    wpart = _clamp01(w_valid.astype(jnp.float32) - (lanef + wfixf))
    zeros = jnp.zeros((L,), jnp.float32)
    lane2 = lax.iota(jnp.int32, L) * SPACING

    def in_copy(k, half):
        b = (2 * k + half) % 2
        i = wid * n_per_w + k
        return pltpu.make_async_copy(
            sp_hbm.at[i, pl.ds(crows * half, crows)],
            img_v.at[b], sem_s.at[b])

    in_copy(0, 0).start()
    pltpu.async_copy(img_v.at[1], dump_hbm, sem_o.at[1])
    pltpu.async_copy(img_v.at[0], dump_hbm, sem_o.at[0])

    def image_body(k, carry):
        i = wid * n_per_w + k
        kn = jnp.minimum(k + 1, n_per_w - 1)
        for half in range(2):
            b = half
            bn = 1 - half
            pltpu.sync_copy(
                pooled_hbm.at[i, pl.ds(cp * half, cp)], pooled_v)
            in_copy(k, half).wait()
            pltpu.make_async_copy(
                img_v.at[bn], dump_hbm, sem_o.at[bn]).wait()
            if half == 0:
                in_copy(k, 1).start()
            else:
                in_copy(kn, 0).start()

            def phase_a(t, _ca):
                for r2 in range(4):
                    h2l = 4 * t + r2
                    ridx = jnp.full((L,), h2l, jnp.int32)
                    for v in range(pw // L):
                        pv = pooled_v[h2l, pl.ds(L * v, L)]
                        mv = _clamp01((pv - 0.5) * 1e30)
                        base = lane2 + (SPACING * L * v)
                        plsc.store_scatter(upm_v, [ridx, base], mv)
                        plsc.store_scatter(upm_v, [ridx, base + 1], mv)
                return _ca

            lax.fori_loop(0, cp // 4, phase_a, 0)

            def phase_b(t, _cb):
                for r2 in range(4):
                    h2l = 4 * t + r2
                    h = SPACING * h2l
                    for jj in range(nvec):
                        mm = upm_v[h2l, pl.ds(L * jj, L)]
                        s0 = img_v[b, h, pl.ds(L * jj, L)]
                        img_v[b, h, pl.ds(L * jj, L)] = s0 * mm
                        s1 = img_v[b, h + 1, pl.ds(L * jj, L)]
                        img_v[b, h + 1, pl.ds(L * jj, L)] = s1 * mm
                return _cb

            lax.fori_loop(0, cp // 4, phase_b, 0)

            hv_c = jnp.minimum(
                jnp.maximum(h_valid - crows * half, 0), crows)

            def wfix_body(h, c1):
                sv = img_v[b, h, pl.ds(L * full_vecs, L)]
                img_v[b, h, pl.ds(L * full_vecs, L)] = sv * wpart

                def vz(jv, c2):
                    img_v[b, h, pl.ds(L * jv, L)] = zeros
                    return c2

                lax.fori_loop(full_vecs + 1, nvec, vz, 0)
                return c1

            lax.fori_loop(0, hv_c * wfix_on, wfix_body, 0)

            def hzero_body(h, c1):
                for jj in range(nvec):
                    img_v[b, h, pl.ds(L * jj, L)] = zeros
                return c1

            lax.fori_loop(hv_c, crows, hzero_body, 0)

            pltpu.async_copy(
                img_v.at[b], out_hbm.at[i, pl.ds(crows * half, crows)],
                sem_o.at[b])
        return carry

    lax.fori_loop(0, n_per_w, image_body, 0)
    pltpu.make_async_copy(img_v.at[0], dump_hbm, sem_o.at[0]).wait()
    pltpu.make_async_copy(img_v.at[1], dump_hbm, sem_o.at[1]).wait()
    pltpu.make_async_copy(sp_hbm.at[0, pl.ds(0, crows)], img_v.at[0],
                          sem_s.at[0]).wait()


@jax.jit
def _unpool(pooled_map, sparse_pattern, original_height, original_width):
    b, c, ph, pw = pooled_map.shape
    oh, ow = sparse_pattern.shape[2], sparse_pattern.shape[3]
    n = b * c
    pooled3 = pooled_map.reshape(n, ph, pw)
    sp3 = sparse_pattern.reshape(n, oh, ow)
    hl16 = jnp.full((L,), jnp.asarray(original_height, jnp.int32))
    wl16 = jnp.full((L,), jnp.asarray(original_width, jnp.int32))

    info = plsc.get_sparse_core_info()
    nc, ns = info.num_cores, info.num_subcores
    nw = nc * ns
    assert n % nw == 0
    n_per_w = n // nw

    body = functools.partial(_sc_body, n_per_w, nc, ph, pw, oh, ow)
    run = pl.kernel(
        body,
        out_type=(
            jax.ShapeDtypeStruct((n, oh, ow), jnp.float32),
            jax.ShapeDtypeStruct((oh // 2, ow), jnp.float32),
        ),
        mesh=plsc.VectorSubcoreMesh(core_axis_name="c", subcore_axis_name="s"),
        compiler_params=pltpu.CompilerParams(needs_layout_passes=False),
        scratch_types=[
            pltpu.VMEM((ph // 2, pw), jnp.float32),
            pltpu.VMEM((2, oh // 2, ow), jnp.float32),
            pltpu.VMEM((ph // 2, ow), jnp.float32),
            pltpu.VMEM((L,), jnp.int32),
            pltpu.VMEM((L,), jnp.int32),
            pltpu.SemaphoreType.DMA((2,)),
            pltpu.SemaphoreType.DMA((2,)),
        ],
    )
    out, _ = run(pooled3, sp3, hl16, wl16)
    return out.reshape(b, c, oh, ow)


def kernel(pooled_map, sparse_pattern, original_height, original_width):
    return _unpool(pooled_map, sparse_pattern, original_height, original_width)


# speedup vs baseline: 4.1371x; 3.2420x over previous
import functools

import jax
import jax.numpy as jnp
from jax.experimental import pallas as pl

SPACING = 2
IMGS_PER_STEP = 48


def _unpool_body(g, eh_ref, ew_ref, pooled_ref, sp_ref, out_ref):
    eh = eh_ref[...]
    ew = ew_ref[...]
    for k in range(g):
        m = (pooled_ref[k] > 0.5).astype(jnp.float32)
        t = jnp.dot(eh, m, preferred_element_type=jnp.float32)
        up = jnp.dot(t, ew, preferred_element_type=jnp.float32)
        out_ref[k] = sp_ref[k] * up


@jax.jit
def _unpool(pooled_map, sparse_pattern, original_height, original_width):
    b, c, ph, pw = pooled_map.shape
    oh, ow = sparse_pattern.shape[2], sparse_pattern.shape[3]
    n = b * c
    pooled3 = pooled_map.reshape(n, ph, pw)
    sp3 = sparse_pattern.reshape(n, oh, ow)

    h_lim = jnp.asarray(original_height, jnp.int32)
    w_lim = jnp.asarray(original_width, jnp.int32)
    i = jax.lax.broadcasted_iota(jnp.int32, (oh, ph), 0)
    j = jax.lax.broadcasted_iota(jnp.int32, (oh, ph), 1)
    eh = ((j == i // SPACING) & (i < h_lim)).astype(jnp.float32)
    jw = jax.lax.broadcasted_iota(jnp.int32, (pw, ow), 0)
    kw = jax.lax.broadcasted_iota(jnp.int32, (pw, ow), 1)
    ew = ((jw == kw // SPACING) & (kw < w_lim)).astype(jnp.float32)

    g = IMGS_PER_STEP
    assert n % g == 0
    body = functools.partial(_unpool_body, g)
    out = pl.pallas_call(
        body,
        grid=(n // g,),
        in_specs=[
            pl.BlockSpec((oh, ph), lambda i: (0, 0)),
            pl.BlockSpec((pw, ow), lambda i: (0, 0)),
            pl.BlockSpec((g, ph, pw), lambda i: (i, 0, 0)),
            pl.BlockSpec((g, oh, ow), lambda i: (i, 0, 0)),
        ],
        out_specs=pl.BlockSpec((g, oh, ow), lambda i: (i, 0, 0)),
        out_shape=jax.ShapeDtypeStruct((n, oh, ow), pooled_map.dtype),
    )(eh, ew, pooled3, sp3)
    return out.reshape(b, c, oh, ow)


def kernel(pooled_map, sparse_pattern, original_height, original_width):
    return _unpool(pooled_map, sparse_pattern, original_height, original_width)


# speedup vs baseline: 4.1640x; 1.0065x over previous
import functools

import jax
import jax.numpy as jnp
from jax.experimental import pallas as pl

SPACING = 2
IMGS_PER_STEP = 32


def _unpool_body(g, eh_ref, ew_ref, pooled_ref, sp_ref, out_ref):
    eh = eh_ref[...]
    ew = ew_ref[...]
    for k in range(g):
        m = (pooled_ref[k] > 0.5).astype(jnp.float32)
        t = jnp.dot(eh, m, preferred_element_type=jnp.float32)
        up = jnp.dot(t, ew, preferred_element_type=jnp.float32)
        out_ref[k] = sp_ref[k] * up


@jax.jit
def _unpool(pooled_map, sparse_pattern, original_height, original_width):
    b, c, ph, pw = pooled_map.shape
    oh, ow = sparse_pattern.shape[2], sparse_pattern.shape[3]
    n = b * c
    pooled3 = pooled_map.reshape(n, ph, pw)
    sp3 = sparse_pattern.reshape(n, oh, ow)

    h_lim = jnp.asarray(original_height, jnp.int32)
    w_lim = jnp.asarray(original_width, jnp.int32)
    i = jax.lax.broadcasted_iota(jnp.int32, (oh, ph), 0)
    j = jax.lax.broadcasted_iota(jnp.int32, (oh, ph), 1)
    eh = ((j == i // SPACING) & (i < h_lim)).astype(jnp.float32)
    jw = jax.lax.broadcasted_iota(jnp.int32, (pw, ow), 0)
    kw = jax.lax.broadcasted_iota(jnp.int32, (pw, ow), 1)
    ew = ((jw == kw // SPACING) & (kw < w_lim)).astype(jnp.float32)

    g = IMGS_PER_STEP
    assert n % g == 0
    body = functools.partial(_unpool_body, g)
    out = pl.pallas_call(
        body,
        grid=(n // g,),
        in_specs=[
            pl.BlockSpec((oh, ph), lambda i: (0, 0)),
            pl.BlockSpec((pw, ow), lambda i: (0, 0)),
            pl.BlockSpec((g, ph, pw), lambda i: (i, 0, 0)),
            pl.BlockSpec((g, oh, ow), lambda i: (i, 0, 0)),
        ],
        out_specs=pl.BlockSpec((g, oh, ow), lambda i: (i, 0, 0)),
        out_shape=jax.ShapeDtypeStruct((n, oh, ow), pooled_map.dtype),
    )(eh, ew, pooled3, sp3)
    return out.reshape(b, c, oh, ow)


def kernel(pooled_map, sparse_pattern, original_height, original_width):
    return _unpool(pooled_map, sparse_pattern, original_height, original_width)
